# baseline (speedup 1.0000x reference)
import functools

import jax
import jax.numpy as jnp
from jax.experimental import pallas as pl
from jax.experimental.pallas import tpu as pltpu

TOP_K = 8


def _router_body(x_ref, wr_ref, g_ref, aux_ref, *, n_experts, n_tokens):
    logits = jnp.dot(x_ref[...], wr_ref[...],
                     preferred_element_type=jnp.float32)
    m = jnp.max(logits, axis=1, keepdims=True)
    ex = jnp.exp(logits - m)
    p_sum = jnp.sum(ex / jnp.sum(ex, axis=1, keepdims=True), axis=0)

    iota = jax.lax.broadcasted_iota(jnp.int32, logits.shape, 1)
    l = logits
    gun = jnp.zeros_like(logits)
    denom = jnp.zeros_like(m)
    top = None
    f_sum = None
    for k in range(TOP_K):
        mk = jnp.max(l, axis=1, keepdims=True)
        idxk = jnp.min(jnp.where(l == mk, iota, n_experts), axis=1,
                       keepdims=True)
        oh = iota == idxk
        if k == 0:
            top = mk
            f_sum = jnp.sum(oh.astype(jnp.float32), axis=0)
        ek = jnp.exp(mk - top)
        gun = gun + jnp.where(oh, ek, 0.0)
        denom = denom + ek
        l = jnp.where(oh, -jnp.inf, l)
    g_ref[...] = gun / denom
    aux = n_experts * jnp.sum(f_sum * p_sum) / (n_tokens * n_tokens)
    aux_ref[...] = aux.reshape(1, 1)


def _ffn_body(x_ref, g_ref, w1_ref, b1_ref, w2_ref, b2_ref, out_ref):
    e = pl.program_id(0)

    @pl.when(e == 0)
    def _():
        out_ref[...] = jnp.zeros_like(out_ref)

    h = jnp.dot(x_ref[...], w1_ref[0], preferred_element_type=jnp.float32)
    h = jax.nn.gelu(h + b1_ref[0]).astype(jnp.bfloat16)
    o = jnp.dot(h, w2_ref[0], preferred_element_type=jnp.float32) + b2_ref[0]
    lane = jax.lax.broadcasted_iota(jnp.int32, g_ref.shape, 1)
    gate_col = jnp.sum(jnp.where(lane == e, g_ref[...], 0.0), axis=1,
                       keepdims=True)
    out_ref[...] += gate_col * o


def kernel(x, W_router, W1, b1, W2, b2):
    B, T, D = x.shape
    E = W_router.shape[1]
    F = W1.shape[2]
    N = B * T
    x2 = x.reshape(N, D)

    router = functools.partial(_router_body, n_experts=E, n_tokens=N)
    G, aux = pl.pallas_call(
        router,
        grid=(1,),
        in_specs=[
            pl.BlockSpec((N, D), lambda i: (0, 0)),
            pl.BlockSpec((D, E), lambda i: (0, 0)),
        ],
        out_specs=[
            pl.BlockSpec((N, E), lambda i: (0, 0)),
            pl.BlockSpec((1, 1), lambda i: (0, 0)),
        ],
        out_shape=[
            jax.ShapeDtypeStruct((N, E), jnp.float32),
            jax.ShapeDtypeStruct((1, 1), jnp.float32),
        ],
    )(x2, W_router)

    out = pl.pallas_call(
        _ffn_body,
        grid=(E,),
        in_specs=[
            pl.BlockSpec((N, D), lambda e: (0, 0)),
            pl.BlockSpec((N, E), lambda e: (0, 0)),
            pl.BlockSpec((1, D, F), lambda e: (e, 0, 0)),
            pl.BlockSpec((1, 1, F), lambda e: (e, 0, 0)),
            pl.BlockSpec((1, F, D), lambda e: (e, 0, 0)),
            pl.BlockSpec((1, 1, D), lambda e: (e, 0, 0)),
        ],
        out_specs=pl.BlockSpec((N, D), lambda e: (0, 0)),
        out_shape=jax.ShapeDtypeStruct((N, D), jnp.float32),
        compiler_params=pltpu.CompilerParams(
            dimension_semantics=("arbitrary",),
        ),
    )(x2.astype(jnp.bfloat16), G, W1.astype(jnp.bfloat16),
      b1.reshape(E, 1, F), W2.astype(jnp.bfloat16), b2.reshape(E, 1, D))
    return out.reshape(B, T, D), aux[0, 0]


# speedup vs baseline: 1.1292x; 1.1292x over previous
import functools

import jax
import jax.numpy as jnp
from jax.experimental import pallas as pl
from jax.experimental.pallas import tpu as pltpu

TOP_K = 8


def _router_body(x_ref, wr_ref, g_ref, aux_ref, *, n_experts, n_tokens):
    logits = jnp.dot(x_ref[...], wr_ref[...],
                     preferred_element_type=jnp.float32)
    m = jnp.max(logits, axis=1, keepdims=True)
    ex = jnp.exp(logits - m)
    p_sum = jnp.sum(ex / jnp.sum(ex, axis=1, keepdims=True), axis=0)

    iota = jax.lax.broadcasted_iota(jnp.int32, logits.shape, 1)
    l = logits
    gun = jnp.zeros_like(logits)
    denom = jnp.zeros_like(m)
    top = None
    f_sum = None
    for k in range(TOP_K):
        mk = jnp.max(l, axis=1, keepdims=True)
        idxk = jnp.min(jnp.where(l == mk, iota, n_experts), axis=1,
                       keepdims=True)
        oh = iota == idxk
        if k == 0:
            top = mk
            f_sum = jnp.sum(oh.astype(jnp.float32), axis=0)
        ek = jnp.exp(mk - top)
        gun = gun + jnp.where(oh, ek, 0.0)
        denom = denom + ek
        l = jnp.where(oh, -jnp.inf, l)
    g_ref[...] = gun / denom
    aux = n_experts * jnp.sum(f_sum * p_sum) / (n_tokens * n_tokens)
    aux_ref[...] = aux.reshape(1, 1)


def _ffn_body(x_ref, g_ref, w1_ref, b1_ref, w2_ref, b2_ref, out_ref):
    e = pl.program_id(0)

    @pl.when(e == 0)
    def _():
        out_ref[...] = jnp.zeros_like(out_ref)

    h = jnp.dot(x_ref[...], w1_ref[0], preferred_element_type=jnp.float32)
    h = jax.nn.gelu(h + b1_ref[0])
    o = jnp.dot(h, w2_ref[0], preferred_element_type=jnp.float32) + b2_ref[0]
    lane = jax.lax.broadcasted_iota(jnp.int32, g_ref.shape, 1)
    gate_col = jnp.sum(jnp.where(lane == e, g_ref[...], 0.0), axis=1,
                       keepdims=True)
    out_ref[...] += gate_col * o


def kernel(x, W_router, W1, b1, W2, b2):
    B, T, D = x.shape
    E = W_router.shape[1]
    F = W1.shape[2]
    N = B * T
    x2 = x.reshape(N, D)

    router = functools.partial(_router_body, n_experts=E, n_tokens=N)
    G, aux = pl.pallas_call(
        router,
        grid=(1,),
        in_specs=[
            pl.BlockSpec((N, D), lambda i: (0, 0)),
            pl.BlockSpec((D, E), lambda i: (0, 0)),
        ],
        out_specs=[
            pl.BlockSpec((N, E), lambda i: (0, 0)),
            pl.BlockSpec((1, 1), lambda i: (0, 0)),
        ],
        out_shape=[
            jax.ShapeDtypeStruct((N, E), jnp.float32),
            jax.ShapeDtypeStruct((1, 1), jnp.float32),
        ],
    )(x2, W_router)

    out = pl.pallas_call(
        _ffn_body,
        grid=(E,),
        in_specs=[
            pl.BlockSpec((N, D), lambda e: (0, 0)),
            pl.BlockSpec((N, E), lambda e: (0, 0)),
            pl.BlockSpec((1, D, F), lambda e: (e, 0, 0)),
            pl.BlockSpec((1, 1, F), lambda e: (e, 0, 0)),
            pl.BlockSpec((1, F, D), lambda e: (e, 0, 0)),
            pl.BlockSpec((1, 1, D), lambda e: (e, 0, 0)),
        ],
        out_specs=pl.BlockSpec((N, D), lambda e: (0, 0)),
        out_shape=jax.ShapeDtypeStruct((N, D), jnp.float32),
        compiler_params=pltpu.CompilerParams(
            dimension_semantics=("arbitrary",),
        ),
    )(x2, G, W1, b1.reshape(E, 1, F), W2, b2.reshape(E, 1, D))
    return out.reshape(B, T, D), aux[0, 0]


# speedup vs baseline: 1.7240x; 1.5267x over previous
import functools

import jax
import jax.numpy as jnp
from jax.experimental import pallas as pl
from jax.experimental.pallas import tpu as pltpu

TOP_K = 8
EG = 2


def _router_body(x_ref, wr_ref, g_ref, aux_ref, *, n_experts, n_tokens):
    logits = jnp.dot(x_ref[...], wr_ref[...],
                     preferred_element_type=jnp.float32)
    m = jnp.max(logits, axis=1, keepdims=True)
    ex = jnp.exp(logits - m)
    p_sum = jnp.sum(ex / jnp.sum(ex, axis=1, keepdims=True), axis=0)

    iota = jax.lax.broadcasted_iota(jnp.int32, logits.shape, 1)
    l = logits
    gun = jnp.zeros_like(logits)
    denom = jnp.zeros_like(m)
    top = None
    f_sum = None
    for k in range(TOP_K):
        mk = jnp.max(l, axis=1, keepdims=True)
        idxk = jnp.min(jnp.where(l == mk, iota, n_experts), axis=1,
                       keepdims=True)
        oh = iota == idxk
        if k == 0:
            top = mk
            f_sum = jnp.sum(oh.astype(jnp.float32), axis=0)
        ek = jnp.exp(mk - top)
        gun = gun + jnp.where(oh, ek, 0.0)
        denom = denom + ek
        l = jnp.where(oh, -jnp.inf, l)
    g_ref[...] = gun / denom
    aux = n_experts * jnp.sum(f_sum * p_sum) / (n_tokens * n_tokens)
    aux_ref[...] = aux.reshape(1, 1)


def _ffn_body(x_ref, g_ref, w1a_ref, w1b_ref, b1_ref,
              w2_ref, b2_ref, out_ref, xb_ref, hg_ref, *, n_experts, ffn_dim):
    i = pl.program_id(0)

    @pl.when(i == 0)
    def _():
        out_ref[...] = jnp.dot(g_ref[...], b2_ref[...],
                               preferred_element_type=jnp.float32)
        xb_ref[...] = x_ref[...].astype(jnp.bfloat16)

    lane_e = jax.lax.broadcasted_iota(jnp.int32, (n_experts, EG), 0)
    lane_j = jax.lax.broadcasted_iota(jnp.int32, (n_experts, EG), 1)
    sel = (lane_e == i * EG + lane_j).astype(jnp.float32)
    gcols = jnp.dot(g_ref[...], sel, preferred_element_type=jnp.float32)

    xb = xb_ref[...]
    for j, w1_ref in enumerate((w1a_ref, w1b_ref)):
        h = jnp.dot(xb, w1_ref[0].astype(jnp.bfloat16),
                    preferred_element_type=jnp.float32)
        h = jax.nn.gelu(h + b1_ref[0, 0, j * ffn_dim:(j + 1) * ffn_dim])
        hg = h * gcols[:, j:j + 1]
        hg_ref[:, j * ffn_dim:(j + 1) * ffn_dim] = hg.astype(jnp.bfloat16)

    out_ref[...] += jnp.dot(hg_ref[...], w2_ref[0].astype(jnp.bfloat16),
                            preferred_element_type=jnp.float32)


def kernel(x, W_router, W1, b1, W2, b2):
    B, T, D = x.shape
    E = W_router.shape[1]
    F = W1.shape[2]
    N = B * T
    x2 = x.reshape(N, D)

    router = functools.partial(_router_body, n_experts=E, n_tokens=N)
    G, aux = pl.pallas_call(
        router,
        grid=(1,),
        in_specs=[
            pl.BlockSpec((N, D), lambda i: (0, 0)),
            pl.BlockSpec((D, E), lambda i: (0, 0)),
        ],
        out_specs=[
            pl.BlockSpec((N, E), lambda i: (0, 0)),
            pl.BlockSpec((1, 1), lambda i: (0, 0)),
        ],
        out_shape=[
            jax.ShapeDtypeStruct((N, E), jnp.float32),
            jax.ShapeDtypeStruct((1, 1), jnp.float32),
        ],
    )(x2, W_router)

    ffn = functools.partial(_ffn_body, n_experts=E, ffn_dim=F)
    out = pl.pallas_call(
        ffn,
        grid=(E // EG,),
        in_specs=[
            pl.BlockSpec((N, D), lambda i: (0, 0)),
            pl.BlockSpec((N, E), lambda i: (0, 0)),
            pl.BlockSpec((1, D, F), lambda i: (EG * i, 0, 0)),
            pl.BlockSpec((1, D, F), lambda i: (EG * i + 1, 0, 0)),
            pl.BlockSpec((1, 1, EG * F), lambda i: (i, 0, 0)),
            pl.BlockSpec((1, EG * F, D), lambda i: (i, 0, 0)),
            pl.BlockSpec((E, D), lambda i: (0, 0)),
        ],
        out_specs=pl.BlockSpec((N, D), lambda i: (0, 0)),
        out_shape=jax.ShapeDtypeStruct((N, D), jnp.float32),
        scratch_shapes=[
            pltpu.VMEM((N, D), jnp.bfloat16),
            pltpu.VMEM((N, EG * F), jnp.bfloat16),
        ],
        compiler_params=pltpu.CompilerParams(
            dimension_semantics=("arbitrary",),
        ),
    )(x2, G, W1, W1, b1.reshape(E // EG, 1, EG * F),
      W2.reshape(E // EG, EG * F, D), b2)
    return out.reshape(B, T, D), aux[0, 0]


# speedup vs baseline: 1.7593x; 1.0205x over previous
import functools

import jax
import jax.numpy as jnp
from jax.experimental import pallas as pl
from jax.experimental.pallas import tpu as pltpu

TOP_K = 8
EG = 4


def _router_body(x_ref, wr_ref, g_ref, aux_ref, *, n_experts, n_tokens):
    logits = jnp.dot(x_ref[...], wr_ref[...],
                     preferred_element_type=jnp.float32)
    m = jnp.max(logits, axis=1, keepdims=True)
    ex = jnp.exp(logits - m)
    p_sum = jnp.sum(ex / jnp.sum(ex, axis=1, keepdims=True), axis=0)

    iota = jax.lax.broadcasted_iota(jnp.int32, logits.shape, 1)
    l = logits
    gun = jnp.zeros_like(logits)
    denom = jnp.zeros_like(m)
    top = None
    f_sum = None
    for k in range(TOP_K):
        mk = jnp.max(l, axis=1, keepdims=True)
        idxk = jnp.min(jnp.where(l == mk, iota, n_experts), axis=1,
                       keepdims=True)
        oh = iota == idxk
        if k == 0:
            top = mk
            f_sum = jnp.sum(oh.astype(jnp.float32), axis=0)
        ek = jnp.exp(mk - top)
        gun = gun + jnp.where(oh, ek, 0.0)
        denom = denom + ek
        l = jnp.where(oh, -jnp.inf, l)
    g_ref[...] = gun / denom
    aux = n_experts * jnp.sum(f_sum * p_sum) / (n_tokens * n_tokens)
    aux_ref[...] = aux.reshape(1, 1)


def _ffn_body(xb_ref, g_ref, w1a_ref, w1b_ref, w1c_ref, w1d_ref, b1_ref,
              w2_ref, b2_ref, out_ref, hg_ref, *, n_experts, ffn_dim):
    i = pl.program_id(0)

    @pl.when(i == 0)
    def _():
        out_ref[...] = jnp.dot(g_ref[...], b2_ref[...],
                               preferred_element_type=jnp.float32)

    lane_e = jax.lax.broadcasted_iota(jnp.int32, (n_experts, EG), 0)
    lane_j = jax.lax.broadcasted_iota(jnp.int32, (n_experts, EG), 1)
    sel = (lane_e == i * EG + lane_j).astype(jnp.float32)
    gcols = jnp.dot(g_ref[...], sel, preferred_element_type=jnp.float32)

    xb = xb_ref[...]
    for j, w1_ref in enumerate((w1a_ref, w1b_ref, w1c_ref, w1d_ref)):
        h = jnp.dot(xb, w1_ref[0].astype(jnp.bfloat16),
                    preferred_element_type=jnp.float32)
        h = jax.nn.gelu(h + b1_ref[0, 0, j * ffn_dim:(j + 1) * ffn_dim])
        hg = h * gcols[:, j:j + 1]
        hg_ref[:, j * ffn_dim:(j + 1) * ffn_dim] = hg.astype(jnp.bfloat16)

    out_ref[...] += jnp.dot(hg_ref[...], w2_ref[0].astype(jnp.bfloat16),
                            preferred_element_type=jnp.float32)


def kernel(x, W_router, W1, b1, W2, b2):
    B, T, D = x.shape
    E = W_router.shape[1]
    F = W1.shape[2]
    N = B * T
    x2 = x.reshape(N, D)

    router = functools.partial(_router_body, n_experts=E, n_tokens=N)
    G, aux = pl.pallas_call(
        router,
        grid=(1,),
        in_specs=[
            pl.BlockSpec((N, D), lambda i: (0, 0)),
            pl.BlockSpec((D, E), lambda i: (0, 0)),
        ],
        out_specs=[
            pl.BlockSpec((N, E), lambda i: (0, 0)),
            pl.BlockSpec((1, 1), lambda i: (0, 0)),
        ],
        out_shape=[
            jax.ShapeDtypeStruct((N, E), jnp.float32),
            jax.ShapeDtypeStruct((1, 1), jnp.float32),
        ],
    )(x2, W_router)

    ffn = functools.partial(_ffn_body, n_experts=E, ffn_dim=F)
    out = pl.pallas_call(
        ffn,
        grid=(E // EG,),
        in_specs=[
            pl.BlockSpec((N, D), lambda i: (0, 0)),
            pl.BlockSpec((N, E), lambda i: (0, 0)),
            pl.BlockSpec((1, D, F), lambda i: (EG * i, 0, 0)),
            pl.BlockSpec((1, D, F), lambda i: (EG * i + 1, 0, 0)),
            pl.BlockSpec((1, D, F), lambda i: (EG * i + 2, 0, 0)),
            pl.BlockSpec((1, D, F), lambda i: (EG * i + 3, 0, 0)),
            pl.BlockSpec((1, 1, EG * F), lambda i: (i, 0, 0)),
            pl.BlockSpec((1, EG * F, D), lambda i: (i, 0, 0)),
            pl.BlockSpec((E, D), lambda i: (0, 0)),
        ],
        out_specs=pl.BlockSpec((N, D), lambda i: (0, 0)),
        out_shape=jax.ShapeDtypeStruct((N, D), jnp.float32),
        scratch_shapes=[
            pltpu.VMEM((N, EG * F), jnp.bfloat16),
        ],
        compiler_params=pltpu.CompilerParams(
            dimension_semantics=("arbitrary",),
        ),
    )(x2.astype(jnp.bfloat16), G, W1, W1, W1, W1,
      b1.reshape(E // EG, 1, EG * F), W2.reshape(E // EG, EG * F, D), b2)
    return out.reshape(B, T, D), aux[0, 0]


# speedup vs baseline: 1.7811x; 1.0124x over previous
import functools

import jax
import jax.numpy as jnp
from jax.experimental import pallas as pl
from jax.experimental.pallas import tpu as pltpu

TOP_K = 8
EG = 4


def _router_body(x_ref, wr_ref, g_ref, aux_ref, *, n_experts, n_tokens):
    logits = jnp.dot(x_ref[...], wr_ref[...],
                     preferred_element_type=jnp.float32)
    m = jnp.max(logits, axis=1, keepdims=True)
    ex = jnp.exp(logits - m)
    p_sum = jnp.sum(ex / jnp.sum(ex, axis=1, keepdims=True), axis=0)

    iota = jax.lax.broadcasted_iota(jnp.int32, logits.shape, 1)
    l = logits
    gun = jnp.zeros_like(logits)
    denom = jnp.zeros_like(m)
    top = None
    f_sum = None
    for k in range(TOP_K):
        mk = jnp.max(l, axis=1, keepdims=True)
        idxk = jnp.min(jnp.where(l == mk, iota, n_experts), axis=1,
                       keepdims=True)
        oh = iota == idxk
        if k == 0:
            top = mk
            f_sum = jnp.sum(oh.astype(jnp.float32), axis=0)
        ek = jnp.exp(mk - top)
        gun = gun + jnp.where(oh, ek, 0.0)
        denom = denom + ek
        l = jnp.where(oh, -jnp.inf, l)
    g_ref[...] = gun / denom
    aux = n_experts * jnp.sum(f_sum * p_sum) / (n_tokens * n_tokens)
    aux_ref[...] = aux.reshape(1, 1)


def _ffn_body(xb_ref, g_ref, w1a_ref, w1b_ref, w1c_ref, w1d_ref, b1_ref,
              w2_ref, b2_ref, out_ref, hg_ref, *, n_experts, ffn_dim):
    i = pl.program_id(0)

    @pl.when(i == 0)
    def _():
        out_ref[...] = jnp.dot(g_ref[...], b2_ref[...],
                               preferred_element_type=jnp.float32)

    lane_e = jax.lax.broadcasted_iota(jnp.int32, (n_experts, EG), 0)
    lane_j = jax.lax.broadcasted_iota(jnp.int32, (n_experts, EG), 1)
    sel = (lane_e == i * EG + lane_j).astype(jnp.float32)
    gcols = jnp.dot(g_ref[...], sel, preferred_element_type=jnp.float32)

    xb = xb_ref[...]
    for j, w1_ref in enumerate((w1a_ref, w1b_ref, w1c_ref, w1d_ref)):
        h = jnp.dot(xb, w1_ref[0].astype(jnp.bfloat16),
                    preferred_element_type=jnp.float32)
        h = jax.nn.gelu(h + b1_ref[0, 0, j * ffn_dim:(j + 1) * ffn_dim])
        hb = h.astype(jnp.bfloat16)
        gb = gcols[:, j:j + 1].astype(jnp.bfloat16)
        hg_ref[:, j * ffn_dim:(j + 1) * ffn_dim] = hb * gb

    out_ref[...] += jnp.dot(hg_ref[...], w2_ref[0].astype(jnp.bfloat16),
                            preferred_element_type=jnp.float32)


def kernel(x, W_router, W1, b1, W2, b2):
    B, T, D = x.shape
    E = W_router.shape[1]
    F = W1.shape[2]
    N = B * T
    x2 = x.reshape(N, D)

    router = functools.partial(_router_body, n_experts=E, n_tokens=N)
    G, aux = pl.pallas_call(
        router,
        grid=(1,),
        in_specs=[
            pl.BlockSpec((N, D), lambda i: (0, 0)),
            pl.BlockSpec((D, E), lambda i: (0, 0)),
        ],
        out_specs=[
            pl.BlockSpec((N, E), lambda i: (0, 0)),
            pl.BlockSpec((1, 1), lambda i: (0, 0)),
        ],
        out_shape=[
            jax.ShapeDtypeStruct((N, E), jnp.float32),
            jax.ShapeDtypeStruct((1, 1), jnp.float32),
        ],
    )(x2, W_router)

    ffn = functools.partial(_ffn_body, n_experts=E, ffn_dim=F)
    out = pl.pallas_call(
        ffn,
        grid=(E // EG,),
        in_specs=[
            pl.BlockSpec((N, D), lambda i: (0, 0)),
            pl.BlockSpec((N, E), lambda i: (0, 0)),
            pl.BlockSpec((1, D, F), lambda i: (EG * i, 0, 0)),
            pl.BlockSpec((1, D, F), lambda i: (EG * i + 1, 0, 0)),
            pl.BlockSpec((1, D, F), lambda i: (EG * i + 2, 0, 0)),
            pl.BlockSpec((1, D, F), lambda i: (EG * i + 3, 0, 0)),
            pl.BlockSpec((1, 1, EG * F), lambda i: (i, 0, 0)),
            pl.BlockSpec((1, EG * F, D), lambda i: (i, 0, 0)),
            pl.BlockSpec((E, D), lambda i: (0, 0)),
        ],
        out_specs=pl.BlockSpec((N, D), lambda i: (0, 0)),
        out_shape=jax.ShapeDtypeStruct((N, D), jnp.float32),
        scratch_shapes=[
            pltpu.VMEM((N, EG * F), jnp.bfloat16),
        ],
        compiler_params=pltpu.CompilerParams(
            dimension_semantics=("arbitrary",),
        ),
    )(x2.astype(jnp.bfloat16), G, W1, W1, W1, W1,
      b1.reshape(E // EG, 1, EG * F), W2.reshape(E // EG, EG * F, D), b2)
    return out.reshape(B, T, D), aux[0, 0]
